# baseline (speedup 1.0000x reference)
import functools

import jax
import jax.numpy as jnp
from jax import lax
from jax.experimental import pallas as pl
from jax.experimental.pallas import tpu as pltpu
from jax.experimental.pallas import tpu_sc as plsc

ROWS = 8192
D = 768
NUM_CORES = 2
ROWS_PER_C = ROWS // NUM_CORES
CHUNK = 1024
NCHUNK = ROWS_PER_C // CHUNK

_mesh = plsc.VectorSubcoreMesh(core_axis_name="c", subcore_axis_name="s")


@functools.partial(
    pl.kernel,
    out_type=jax.ShapeDtypeStruct((ROWS, D), jnp.float32),
    mesh=_mesh,
    scratch_types=(
        [pltpu.VMEM_SHARED((CHUNK, D), jnp.float32) for _ in range(2)]
        + [pltpu.SemaphoreType.DMA for _ in range(2 * NCHUNK)]
    ),
)
def _pe_linear_gather(pe_hbm, out_hbm, *scratch):
    bufs = scratch[:2]
    in_sems = scratch[2 : 2 + NCHUNK]
    out_sems = scratch[2 + NCHUNK :]
    cid = lax.axis_index("c")
    base = cid * ROWS_PER_C

    @pl.when(lax.axis_index("s") == 0)
    def _():
        def slab(i):
            return pl.ds(base + i * CHUNK, CHUNK)

        def load(i):
            return pltpu.async_copy(pe_hbm.at[slab(i)], bufs[i % 2], in_sems[i])

        def store(i):
            return pltpu.async_copy(bufs[i % 2], out_hbm.at[slab(i)], out_sems[i])

        loads = [load(0)]
        stores = []
        for i in range(NCHUNK):
            loads[i].wait()
            stores.append(store(i))
            if i + 1 < NCHUNK:
                if i - 1 >= 0:
                    stores[i - 1].wait()
                loads.append(load(i + 1))
        stores[NCHUNK - 2].wait()
        stores[NCHUNK - 1].wait()


def kernel(x, pe):
    del x
    return _pe_linear_gather(pe)


# speedup vs baseline: 1.0065x; 1.0065x over previous
import functools

import jax
import jax.numpy as jnp
from jax import lax
from jax.experimental import pallas as pl
from jax.experimental.pallas import tpu as pltpu
from jax.experimental.pallas import tpu_sc as plsc

ROWS = 8192
D = 768

NUM_WORKERS = 32
STREAM_ROWS = 5120
ROWS_PER_W = STREAM_ROWS // NUM_WORKERS
CHUNK = 40
NCHUNK = ROWS_PER_W // CHUNK
NBUF = 2

NUM_CORES = 2
SP_ROWS = ROWS - STREAM_ROWS
SP_ROWS_PER_C = SP_ROWS // NUM_CORES
SP_CHUNK = 512
SP_NCHUNK = SP_ROWS_PER_C // SP_CHUNK

_mesh = plsc.VectorSubcoreMesh(core_axis_name="c", subcore_axis_name="s")


@functools.partial(
    pl.kernel,
    out_type=jax.ShapeDtypeStruct((ROWS, D), jnp.float32),
    mesh=_mesh,
    scratch_types=(
        [pltpu.VMEM((CHUNK, D), jnp.float32) for _ in range(NBUF)]
        + [pltpu.VMEM_SHARED((SP_CHUNK, D), jnp.float32) for _ in range(2)]
        + [pltpu.SemaphoreType.DMA for _ in range(2 * NCHUNK + 2 * SP_NCHUNK)]
    ),
)
def _pe_linear_gather(pe_hbm, out_hbm, *scratch):
    bufs = scratch[:NBUF]
    sp_bufs = scratch[NBUF : NBUF + 2]
    sems = scratch[NBUF + 2 :]
    in_sems = sems[:NCHUNK]
    out_sems = sems[NCHUNK : 2 * NCHUNK]
    sp_in_sems = sems[2 * NCHUNK : 2 * NCHUNK + SP_NCHUNK]
    sp_out_sems = sems[2 * NCHUNK + SP_NCHUNK :]
    cid = lax.axis_index("c")
    sid = lax.axis_index("s")

    sp_base = STREAM_ROWS + cid * SP_ROWS_PER_C

    @pl.when(sid == 0)
    def _():
        def slab(i):
            return pl.ds(sp_base + i * SP_CHUNK, SP_CHUNK)

        def load(i):
            return pltpu.async_copy(pe_hbm.at[slab(i)], sp_bufs[i % 2], sp_in_sems[i])

        def store(i):
            return pltpu.async_copy(sp_bufs[i % 2], out_hbm.at[slab(i)], sp_out_sems[i])

        loads = [load(0)]
        stores = []
        for i in range(SP_NCHUNK):
            loads[i].wait()
            stores.append(store(i))
            if i + 1 < SP_NCHUNK:
                if i - 1 >= 0:
                    stores[i - 1].wait()
                loads.append(load(i + 1))
        stores[SP_NCHUNK - 2].wait()
        stores[SP_NCHUNK - 1].wait()

    wid = sid * 2 + cid
    base = wid * ROWS_PER_W

    def slab(i):
        return pl.ds(base + i * CHUNK, CHUNK)

    def load(i):
        return pltpu.async_copy(pe_hbm.at[slab(i)], bufs[i % NBUF], in_sems[i])

    def store(i):
        return pltpu.async_copy(bufs[i % NBUF], out_hbm.at[slab(i)], out_sems[i])

    loads = [load(i) for i in range(NBUF)]
    stores = []
    for i in range(NCHUNK):
        loads[i].wait()
        stores.append(store(i))
        if i + NBUF < NCHUNK:
            stores[i].wait()
            loads.append(load(i + NBUF))
    for i in range(max(0, NCHUNK - NBUF), NCHUNK):
        stores[i].wait()


def kernel(x, pe):
    del x
    return _pe_linear_gather(pe)


# speedup vs baseline: 1.1673x; 1.1598x over previous
import functools

import jax
import jax.numpy as jnp
from jax import lax
from jax.experimental import pallas as pl
from jax.experimental.pallas import tpu as pltpu
from jax.experimental.pallas import tpu_sc as plsc

ROWS = 8192
D = 768
NUM_WORKERS = 32
ROWS_PER_W = ROWS // NUM_WORKERS
CHUNK = 32
NCHUNK = ROWS_PER_W // CHUNK
NBUF = 5

_mesh = plsc.VectorSubcoreMesh(core_axis_name="c", subcore_axis_name="s")


@functools.partial(
    pl.kernel,
    out_type=jax.ShapeDtypeStruct((ROWS, D), jnp.float32),
    mesh=_mesh,
    scratch_types=(
        [pltpu.VMEM((CHUNK, D), jnp.float32) for _ in range(NBUF)]
        + [pltpu.SemaphoreType.DMA for _ in range(2 * NCHUNK)]
    ),
)
def _pe_linear_gather(pe_hbm, out_hbm, *scratch):
    bufs = scratch[:NBUF]
    in_sems = scratch[NBUF : NBUF + NCHUNK]
    out_sems = scratch[NBUF + NCHUNK :]
    wid = lax.axis_index("s") * 2 + lax.axis_index("c")
    base = wid * ROWS_PER_W

    def slab(i):
        return pl.ds(base + i * CHUNK, CHUNK)

    def load(i):
        return pltpu.async_copy(pe_hbm.at[slab(i)], bufs[i % NBUF], in_sems[i])

    def store(i):
        return pltpu.async_copy(bufs[i % NBUF], out_hbm.at[slab(i)], out_sems[i])

    loads = [load(i) for i in range(NBUF)]
    stores = []
    for i in range(NCHUNK):
        loads[i].wait()
        stores.append(store(i))
        if i + NBUF < NCHUNK:
            stores[i].wait()
            loads.append(load(i + NBUF))
    for i in range(max(0, NCHUNK - NBUF), NCHUNK):
        stores[i].wait()


def kernel(x, pe):
    del x
    return _pe_linear_gather(pe)
